# baseline (speedup 1.0000x reference)
import jax
import jax.numpy as jnp
from jax import lax
from jax.experimental import pallas as pl
from jax.experimental.pallas import tpu as pltpu
from jax.experimental.pallas import tpu_sc as plsc

VOCAB = 100000
SEQ = 200
DIM = 128
BATCH = 1024

_INFO = plsc.get_sparse_core_info()
_NC = _INFO.num_cores
_NS = _INFO.num_subcores
_NW = _NC * _NS
_PER_W = BATCH * SEQ // _NW

_CHUNK = 128
_NCHUNK = _PER_W // _CHUNK
_LOOKAHEAD = 3
_NBUF = 5

_LANES = 16
_VECS_PER_LINE = DIM // _LANES


def _body(x_hbm, tok_hbm, pos_hbm, out_hbm,
          idx_v, pos_v, buf0, buf1, buf2, buf3, buf4,
          g0, g1, g2, g3, g4, s0, s1, s2, s3, s4):
    bufs = (buf0, buf1, buf2, buf3, buf4)
    gsems = (g0, g1, g2, g3, g4)
    ssems = (s0, s1, s2, s3, s4)

    wid = lax.axis_index("s") * _NC + lax.axis_index("c")
    base0 = wid * _PER_W

    pltpu.sync_copy(pos_hbm, pos_v)
    pltpu.sync_copy(x_hbm.at[pl.ds(base0, _PER_W)], idx_v)

    def gather(c, bi):
        pltpu.async_copy(
            tok_hbm.at[idx_v.at[pl.ds(c * _CHUNK, _CHUNK)]],
            bufs[bi], gsems[bi])

    def wait_gather(c, bi):
        pltpu.make_async_copy(
            tok_hbm.at[idx_v.at[pl.ds(c * _CHUNK, _CHUNK)]],
            bufs[bi], gsems[bi]).wait()

    def store(c, bi):
        pltpu.async_copy(
            bufs[bi], out_hbm.at[pl.ds(base0 + c * _CHUNK, _CHUNK)],
            ssems[bi])

    def wait_store(c, bi):
        pltpu.make_async_copy(
            bufs[bi], out_hbm.at[pl.ds(base0 + c * _CHUNK, _CHUNK)],
            ssems[bi]).wait()

    for c in range(_LOOKAHEAD):
        gather(c, c % _NBUF)

    @pl.loop(0, _NCHUNK, step=_NBUF)
    def _block(c0):
        for b in range(_NBUF):
            c = c0 + b
            g = c + _LOOKAHEAD
            bg = (b + _LOOKAHEAD) % _NBUF

            @pl.when(g < _NCHUNK)
            def _issue():
                @pl.when(c >= _NBUF - _LOOKAHEAD)
                def _drain():
                    wait_store(c - (_NBUF - _LOOKAHEAD), bg)
                gather(g, bg)

            wait_gather(c, b)
            buf = bufs[b]
            p0 = lax.rem(c * _CHUNK, SEQ)

            @pl.loop(0, _CHUNK)
            def _line(l):
                p = p0 + l
                p = jnp.where(p >= SEQ, p - SEQ, p)
                for j in range(_VECS_PER_LINE):
                    sl = pl.ds(j * _LANES, _LANES)
                    buf[l, sl] = buf[l, sl] + pos_v[p, sl]

            store(c, b)

    for c in range(_NCHUNK - _NBUF, _NCHUNK):
        wait_store(c, c % _NBUF)


@jax.jit
def _run(x_flat, token_table, position_table):
    mesh = plsc.VectorSubcoreMesh(core_axis_name="c", subcore_axis_name="s")
    return pl.kernel(
        _body,
        out_type=jax.ShapeDtypeStruct((BATCH * SEQ, DIM), jnp.float32),
        mesh=mesh,
        scratch_types=(
            [pltpu.VMEM((_PER_W,), jnp.int32),
             pltpu.VMEM((SEQ, DIM), jnp.float32)]
            + [pltpu.VMEM((_CHUNK, DIM), jnp.float32)] * _NBUF
            + [pltpu.SemaphoreType.DMA] * (2 * _NBUF)
        ),
    )(x_flat, token_table, position_table)


def kernel(x, token_table, position_table):
    x_flat = x.reshape(-1).astype(jnp.int32)
    out = _run(x_flat, token_table, position_table)
    return out.reshape(x.shape[0], x.shape[1], DIM)


# speedup vs baseline: 2.5449x; 2.5449x over previous
import jax
import jax.numpy as jnp
from jax import lax
from jax.experimental import pallas as pl
from jax.experimental.pallas import tpu as pltpu
from jax.experimental.pallas import tpu_sc as plsc

VOCAB = 100000
SEQ = 200
DIM = 128
BATCH = 1024

_INFO = plsc.get_sparse_core_info()
_NC = _INFO.num_cores
_NS = _INFO.num_subcores
_NW = _NC * _NS
_ROWS_PER_W = BATCH // _NW
_PER_W = _ROWS_PER_W * SEQ

_UNITS = []
for _r in range(_ROWS_PER_W):
    _UNITS.append((_r * SEQ, 104, 0))
    _UNITS.append((_r * SEQ + 104, 96, 104))
_NU = len(_UNITS)

_NBUF = 6
_LOOKAHEAD = 3
_BUF_LINES = 104

_LANES = 16
_VECS_PER_LINE = DIM // _LANES


def _body(x_hbm, tok_hbm, pos_hbm, out_hbm,
          idx_v, pos_v, buf0, buf1, buf2, buf3, buf4, buf5,
          g0, g1, g2, g3, g4, g5, s0, s1, s2, s3, s4, s5):
    bufs = (buf0, buf1, buf2, buf3, buf4, buf5)
    gsems = (g0, g1, g2, g3, g4, g5)
    ssems = (s0, s1, s2, s3, s4, s5)

    wid = lax.axis_index("s") * _NC + lax.axis_index("c")
    base0 = wid * _PER_W

    pltpu.sync_copy(pos_hbm, pos_v)
    pltpu.sync_copy(x_hbm.at[pl.ds(base0, _PER_W)], idx_v)

    def gather(u):
        off, n, _ = _UNITS[u]
        bi = u % _NBUF
        pltpu.async_copy(
            tok_hbm.at[idx_v.at[pl.ds(off, n)]],
            bufs[bi].at[pl.ds(0, n), :], gsems[bi])

    def wait_gather(u):
        off, n, _ = _UNITS[u]
        bi = u % _NBUF
        pltpu.make_async_copy(
            tok_hbm.at[idx_v.at[pl.ds(off, n)]],
            bufs[bi].at[pl.ds(0, n), :], gsems[bi]).wait()

    def store(u):
        off, n, _ = _UNITS[u]
        bi = u % _NBUF
        pltpu.async_copy(
            bufs[bi].at[pl.ds(0, n), :],
            out_hbm.at[pl.ds(base0 + off, n)], ssems[bi])

    def wait_store(u):
        off, n, _ = _UNITS[u]
        bi = u % _NBUF
        pltpu.make_async_copy(
            bufs[bi].at[pl.ds(0, n), :],
            out_hbm.at[pl.ds(base0 + off, n)], ssems[bi]).wait()

    for u in range(_LOOKAHEAD):
        gather(u)
    for u in range(_NU):
        g = u + _LOOKAHEAD
        if g < _NU:
            if u >= _NBUF - _LOOKAHEAD:
                wait_store(u - (_NBUF - _LOOKAHEAD))
            gather(g)
        wait_gather(u)
        buf = bufs[u % _NBUF]
        _, n, pb = _UNITS[u]

        @pl.loop(0, n)
        def _line(l):
            for j in range(_VECS_PER_LINE):
                sl = pl.ds(j * _LANES, _LANES)
                buf[l, sl] = buf[l, sl] + pos_v[pb + l, sl]

        store(u)
    for u in range(_NU - _NBUF, _NU):
        wait_store(u)


@jax.jit
def _run(x_flat, token_table, position_table):
    mesh = plsc.VectorSubcoreMesh(core_axis_name="c", subcore_axis_name="s")
    return pl.kernel(
        _body,
        out_type=jax.ShapeDtypeStruct((BATCH * SEQ, DIM), jnp.float32),
        mesh=mesh,
        scratch_types=(
            [pltpu.VMEM((_PER_W,), jnp.int32),
             pltpu.VMEM((SEQ, DIM), jnp.float32)]
            + [pltpu.VMEM((_BUF_LINES, DIM), jnp.float32)] * _NBUF
            + [pltpu.SemaphoreType.DMA] * (2 * _NBUF)
        ),
    )(x_flat, token_table, position_table)


def kernel(x, token_table, position_table):
    x_flat = x.reshape(-1).astype(jnp.int32)
    out = _run(x_flat, token_table, position_table)
    return out.reshape(x.shape[0], x.shape[1], DIM)


# speedup vs baseline: 2.5533x; 1.0033x over previous
import jax
import jax.numpy as jnp
from jax import lax
from jax.experimental import pallas as pl
from jax.experimental.pallas import tpu as pltpu
from jax.experimental.pallas import tpu_sc as plsc

VOCAB = 100000
SEQ = 200
DIM = 128
BATCH = 1024

_INFO = plsc.get_sparse_core_info()
_NC = _INFO.num_cores
_NS = _INFO.num_subcores
_NW = _NC * _NS
_ROWS_PER_W = BATCH // _NW
_PER_W = _ROWS_PER_W * SEQ

_CHUNKS = ((0, 104), (104, 96))

_LANES = 16
_VECS_PER_LINE = DIM // _LANES
_NBUF = 3


def _body(x_hbm, tok_hbm, pos_hbm, out_hbm,
          idx_v, pos_v, buf0, buf1, buf2,
          psem, g0, g1, g2, s0, s1, s2):
    bufs = (buf0, buf1, buf2)
    gsems = (g0, g1, g2)
    ssems = (s0, s1, s2)

    wid = lax.axis_index("s") * _NC + lax.axis_index("c")
    row0 = wid * _ROWS_PER_W
    base0 = row0 * SEQ

    pltpu.sync_copy(x_hbm.at[pl.ds(base0, _PER_W)], idx_v)
    pos_copy = pltpu.make_async_copy(pos_hbm, pos_v, psem)
    pos_copy.start()

    def gather(r):
        b = r % _NBUF
        for off, n in _CHUNKS:
            pltpu.async_copy(
                tok_hbm.at[idx_v.at[pl.ds(r * SEQ + off, n)]],
                bufs[b].at[pl.ds(off, n), :],
                gsems[b],
            )

    def wait_gather(r):
        b = r % _NBUF
        for off, n in _CHUNKS:
            pltpu.make_async_copy(
                tok_hbm.at[idx_v.at[pl.ds(r * SEQ + off, n)]],
                bufs[b].at[pl.ds(off, n), :],
                gsems[b],
            ).wait()

    def store_half(r, off, n):
        b = r % _NBUF
        pltpu.async_copy(
            bufs[b].at[pl.ds(off, n), :],
            out_hbm.at[pl.ds(base0 + r * SEQ + off, n)],
            ssems[b],
        )

    def wait_store(r):
        b = r % _NBUF
        for off, n in _CHUNKS:
            pltpu.make_async_copy(
                bufs[b].at[pl.ds(off, n), :],
                out_hbm.at[pl.ds(base0 + r * SEQ + off, n)],
                ssems[b],
            ).wait()

    gather(0)
    pos_copy.wait()
    for r in range(_ROWS_PER_W):
        if r + 1 < _ROWS_PER_W:
            if r >= 2:
                wait_store(r - 2)
            gather(r + 1)
        wait_gather(r)
        buf = bufs[r % _NBUF]

        for off, n in _CHUNKS:
            @pl.loop(0, n)
            def _line(l):
                for j in range(_VECS_PER_LINE):
                    sl = pl.ds(j * _LANES, _LANES)
                    buf[off + l, sl] = buf[off + l, sl] + pos_v[off + l, sl]

            store_half(r, off, n)
    for r in range(_ROWS_PER_W - 3, _ROWS_PER_W):
        wait_store(r)


@jax.jit
def _run(x_flat, token_table, position_table):
    mesh = plsc.VectorSubcoreMesh(core_axis_name="c", subcore_axis_name="s")
    return pl.kernel(
        _body,
        out_type=jax.ShapeDtypeStruct((BATCH * SEQ, DIM), jnp.float32),
        mesh=mesh,
        scratch_types=(
            [pltpu.VMEM((_PER_W,), jnp.int32),
             pltpu.VMEM((SEQ, DIM), jnp.float32)]
            + [pltpu.VMEM((SEQ, DIM), jnp.float32)] * _NBUF
            + [pltpu.SemaphoreType.DMA] * (1 + 2 * _NBUF)
        ),
    )(x_flat, token_table, position_table)


def kernel(x, token_table, position_table):
    x_flat = x.reshape(-1).astype(jnp.int32)
    out = _run(x_flat, token_table, position_table)
    return out.reshape(x.shape[0], x.shape[1], DIM)
